# Initial kernel scaffold; baseline (speedup 1.0000x reference)
import jax
import jax.numpy as jnp
from jax.experimental import pallas as pl


def kernel(token_ids, weight):
    raise NotImplementedError("write your pallas kernel here")


# speedup vs baseline: 3.3330x; 3.3330x over previous
import jax
import jax.numpy as jnp
from jax import lax
from jax.experimental import pallas as pl
from jax.experimental.pallas import tpu as pltpu
from jax.experimental.pallas import tpu_sc as plsc

_info = plsc.get_sparse_core_info()
NC, NS = _info.num_cores, _info.num_subcores
NW = NC * NS

B = 4096 * 50
D = 128
ROWS_PER_W = B // NW
CH = 128
NCH = ROWS_PER_W // CH
NBUF = 2


def _body(tok_hbm, w_hbm, out_hbm, idx_v, rows_v, *sems):
  wid = lax.axis_index("s") * NC + lax.axis_index("c")
  base = wid * ROWS_PER_W

  pltpu.sync_copy(tok_hbm.at[wid], idx_v)

  def start_gather(c, b):
    pltpu.async_copy(w_hbm.at[idx_v.at[c]], rows_v.at[b], sems[b])

  for b in range(NBUF):
    start_gather(b, b)

  @pl.loop(0, NCH, step=NBUF)
  def _(g0):
    for b in range(NBUF):
      g = g0 + b
      pltpu.make_async_copy(w_hbm.at[idx_v.at[g]], rows_v.at[b], sems[b]).wait()
      pltpu.sync_copy(rows_v.at[b], out_hbm.at[pl.ds(base + g * CH, CH)])
      nxt = g + NBUF

      @pl.when(nxt < NCH)
      def _():
        start_gather(nxt, b)


@jax.jit
def kernel(token_ids, weight):
  tok = token_ids.reshape(NW, NCH, CH).astype(jnp.int32)
  mesh = plsc.VectorSubcoreMesh(core_axis_name="c", subcore_axis_name="s")
  call = pl.kernel(
      _body,
      out_type=jax.ShapeDtypeStruct((B, D), jnp.float32),
      mesh=mesh,
      scratch_types=[
          pltpu.VMEM((NCH, CH), jnp.int32),
          pltpu.VMEM((NBUF, CH, D), jnp.float32),
      ] + [pltpu.SemaphoreType.DMA] * NBUF,
  )
  out = call(tok, weight)
  return out.reshape(token_ids.shape[0], token_ids.shape[1], D)


# speedup vs baseline: 3.3521x; 1.0057x over previous
import jax
import jax.numpy as jnp
from jax import lax
from jax.experimental import pallas as pl
from jax.experimental.pallas import tpu as pltpu
from jax.experimental.pallas import tpu_sc as plsc

_info = plsc.get_sparse_core_info()
NC, NS = _info.num_cores, _info.num_subcores
NW = NC * NS

B = 4096 * 50
D = 128
ROWS_PER_W = B // NW
CH = 128
NCH = ROWS_PER_W // CH
NBUF = 5


def _body(tok_hbm, w_hbm, out_hbm, idx_v, rows_v, *sems):
  gsems, wsems = sems[:NBUF], sems[NBUF:]
  wid = lax.axis_index("s") * NC + lax.axis_index("c")
  base = wid * ROWS_PER_W

  pltpu.sync_copy(tok_hbm.at[wid], idx_v)

  def start_gather(c, b):
    pltpu.async_copy(w_hbm.at[idx_v.at[c]], rows_v.at[b], gsems[b])

  def wb_desc(c, b):
    return pltpu.make_async_copy(
        rows_v.at[b], out_hbm.at[pl.ds(base + c * CH, CH)], wsems[b])

  for b in range(NBUF):
    start_gather(b, b)

  @pl.loop(0, NCH, step=NBUF)
  def _(g0):
    for b in range(NBUF):
      g = g0 + b
      pltpu.make_async_copy(w_hbm.at[idx_v.at[g]], rows_v.at[b], gsems[b]).wait()
      wb_desc(g, b).start()
      pb = (b - 1) % NBUF

      @pl.when(g > 0)
      def _():
        wb_desc(g - 1, pb).wait()
      nxt = g - 1 + NBUF

      @pl.when((g > 0) & (nxt < NCH))
      def _():
        start_gather(nxt, pb)

  wb_desc(NCH - 1, (NCH - 1) % NBUF).wait()


@jax.jit
def kernel(token_ids, weight):
  tok = token_ids.reshape(NW, NCH, CH).astype(jnp.int32)
  mesh = plsc.VectorSubcoreMesh(core_axis_name="c", subcore_axis_name="s")
  call = pl.kernel(
      _body,
      out_type=jax.ShapeDtypeStruct((B, D), jnp.float32),
      mesh=mesh,
      scratch_types=[
          pltpu.VMEM((NCH, CH), jnp.int32),
          pltpu.VMEM((NBUF, CH, D), jnp.float32),
      ] + [pltpu.SemaphoreType.DMA] * (2 * NBUF),
  )
  out = call(tok, weight)
  return out.reshape(token_ids.shape[0], token_ids.shape[1], D)
